# baseline (speedup 1.0000x reference)
import functools

import jax
import jax.numpy as jnp
from jax import lax
from jax.experimental import pallas as pl
from jax.experimental.pallas import tpu as pltpu
from jax.experimental.pallas import tpu_sc as plsc

_N = 10000
_DF = 128
_H = 128
_OUTD = 128
_E = 320000
_ES = 320000
_ENS = 160000
_ESN = 160000
_EDGE_F = 16
_DEPTH = 2
_NSG = 512
_NB = 64

_BM = 400
_GRID = _N // _BM
_NC = 2
_NS = 16
_NW = _NC * _NS
_CH = 128
_RPT = 632
_NP = _RPT * _NS

_f32 = jnp.float32


_ZB = 64


def _zero_stripe(zbuf, dst, r0):
  nfull = _RPT // _ZB
  rz = _RPT % _ZB
  for q in range(nfull):
    pltpu.sync_copy(zbuf, dst.at[pl.ds(pl.multiple_of(r0 + q * _ZB, 8), _ZB)])
  if rz:
    pltpu.sync_copy(zbuf.at[pl.ds(0, rz)],
                    dst.at[pl.ds(pl.multiple_of(r0 + nfull * _ZB, 8), rz)])


def _flush_stripe(acc, out_hbm, r0, obase):
  nfull = _RPT // 128
  rz = _RPT % 128
  for q in range(nfull):
    pltpu.sync_copy(acc.at[pl.ds(pl.multiple_of(r0 + q * 128, 8), 128)],
                    out_hbm.at[pl.ds(pl.multiple_of(obase + r0 + q * 128, 8),
                                     128)])
  if rz:
    pltpu.sync_copy(
        acc.at[pl.ds(pl.multiple_of(r0 + nfull * 128, 8), rz)],
        out_hbm.at[pl.ds(pl.multiple_of(obase + r0 + nfull * 128, 8), rz)])


def _scatter_sets(out_hbm, sets, acc, zbuf, sidx, didx, rows,
                  sidx16, didx16, rows16, sidx8, didx8, rows8,
                  isem, gsem, ssem, c, s):
  r0 = s * _RPT

  for (table_hbm, se, de, cnt, obase) in sets:
    per = cnt // _NW
    base = c * (cnt // _NC) + s * per
    nfull = per // _CH
    rem = per % _CH
    nq = nfull // 4
    leftovers = list(range(4 * nq, nfull))

    def idx_load(j, ib):
      off = pl.multiple_of(base + j * _CH, 8)
      pltpu.async_copy(se.at[pl.ds(off, _CH)], sidx[ib], isem[ib])
      pltpu.async_copy(de.at[pl.ds(off, _CH)], didx[ib], isem[ib])

    def idx_wait(j, ib):
      off = pl.multiple_of(base + j * _CH, 8)
      pltpu.make_async_copy(se.at[pl.ds(off, _CH)], sidx[ib], isem[ib]).wait()
      pltpu.make_async_copy(de.at[pl.ds(off, _CH)], didx[ib], isem[ib]).wait()

    def gather(ib, rb):
      pltpu.async_copy(table_hbm.at[sidx[ib]], rows[rb], gsem[rb])

    def gather_wait(ib, rb):
      pltpu.make_async_copy(table_hbm.at[sidx[ib]], rows[rb], gsem[rb]).wait()

    def scat(ib, rb):
      pltpu.async_copy(rows[rb], acc.at[didx[ib]], ssem[rb], add=True)

    def scat_wait(ib, rb):
      pltpu.make_async_copy(rows[rb], acc.at[didx[ib]], ssem[rb]).wait()

    _zero_stripe(zbuf, acc, r0)
    plsc.subcore_barrier()

    idx_load(0, 0)
    idx_load(1, 1)

    def quad(q, carry):
      jq = 4 * q
      for b in range(4):
        j = jq + b
        ib = b
        rb = b % 2
        pib = (b + 3) % 4
        prb = (b + 1) % 2
        if b >= 2:
          scat_wait(b - 2, rb)
        else:
          @pl.when(q >= 1)
          def _():
            scat_wait((b + 2) % 4, rb)

        @pl.when(j + 2 < nfull)
        def _():
          idx_load(j + 2, (b + 2) % 4)

        idx_wait(j, ib)
        gather(ib, rb)
        if b == 0:
          @pl.when(q >= 1)
          def _():
            gather_wait(pib, prb)
            scat(pib, prb)
        else:
          gather_wait(pib, prb)
          scat(pib, prb)
      return carry

    lax.fori_loop(0, nq, quad, 0)

    for j in leftovers:
      ib = j % 4
      rb = j % 2
      if j >= 2:
        scat_wait((j - 2) % 4, rb)
      if j + 2 < nfull:
        idx_load(j + 2, (j + 2) % 4)
      idx_wait(j, ib)
      gather(ib, rb)
      if j >= 1:
        gather_wait((j - 1) % 4, (j - 1) % 2)
        scat((j - 1) % 4, (j - 1) % 2)
    gather_wait((nfull - 1) % 4, (nfull - 1) % 2)
    scat((nfull - 1) % 4, (nfull - 1) % 2)

    if rem:
      off = pl.multiple_of(base + nfull * _CH, 8)
      si, di, rw = (sidx16, didx16, rows16) if rem == 16 else (sidx8, didx8, rows8)
      pltpu.sync_copy(se.at[pl.ds(off, rem)], si)
      pltpu.sync_copy(de.at[pl.ds(off, rem)], di)
      pltpu.async_copy(table_hbm.at[si], rw, gsem[0]).wait()
      pltpu.sync_copy(rw, acc.at[di], add=True)

    scat_wait((nfull - 2) % 4, (nfull - 2) % 2)
    scat_wait((nfull - 1) % 4, (nfull - 1) % 2)

    plsc.subcore_barrier()
    _flush_stripe(acc, out_hbm, r0, obase)
    plsc.subcore_barrier()


def _sc_scatter5_body(h_hbm, s1, d1, s2, d2, s3, d3, s4, d4, eap_hbm, eidx,
                      z_hbm, out_hbm,
                      acc, zbuf, si0, si1, si2, si3, di0, di1, di2, di3,
                      rows0, rows1, sidx16, didx16, rows16, sidx8, didx8,
                      rows8, is0, is1, is2, is3, gsem0, gsem1, ssem0, ssem1):
  c = lax.axis_index("c")
  s = lax.axis_index("s")
  pltpu.sync_copy(z_hbm, zbuf)
  sets = ((h_hbm, s1, d1, _E, (c * 4 + 0) * _NP),
          (h_hbm, s2, d2, _ES, (c * 4 + 1) * _NP),
          (h_hbm, s3, d3, _ENS, (c * 4 + 2) * _NP),
          (h_hbm, s4, d4, _ESN, (c * 4 + 3) * _NP),
          (eap_hbm, eidx, d1, _E, (8 + c) * _NP))
  _scatter_sets(out_hbm, sets, acc, zbuf, (si0, si1, si2, si3),
                (di0, di1, di2, di3), (rows0, rows1),
                sidx16, didx16, rows16, sidx8, didx8, rows8,
                (is0, is1, is2, is3), (gsem0, gsem1), (ssem0, ssem1), c, s)


def _sc_scatter4_body(h_hbm, s1, d1, s2, d2, s3, d3, s4, d4, z_hbm, out_hbm,
                      acc, zbuf, si0, si1, si2, si3, di0, di1, di2, di3,
                      rows0, rows1, sidx16, didx16, rows16, sidx8, didx8,
                      rows8, is0, is1, is2, is3, gsem0, gsem1, ssem0, ssem1):
  c = lax.axis_index("c")
  s = lax.axis_index("s")
  pltpu.sync_copy(z_hbm, zbuf)
  sets = ((h_hbm, s1, d1, _E, (c * 4 + 0) * _NP),
          (h_hbm, s2, d2, _ES, (c * 4 + 1) * _NP),
          (h_hbm, s3, d3, _ENS, (c * 4 + 2) * _NP),
          (h_hbm, s4, d4, _ESN, (c * 4 + 3) * _NP))
  _scatter_sets(out_hbm, sets, acc, zbuf, (si0, si1, si2, si3),
                (di0, di1, di2, di3), (rows0, rows1),
                sidx16, didx16, rows16, sidx8, didx8, rows8,
                (is0, is1, is2, is3), (gsem0, gsem1), (ssem0, ssem1), c, s)


_SCR = [
    pltpu.VMEM_SHARED((_NP, _H), _f32),
    pltpu.VMEM((_ZB, _H), _f32),
    pltpu.VMEM((_CH,), jnp.int32),
    pltpu.VMEM((_CH,), jnp.int32),
    pltpu.VMEM((_CH,), jnp.int32),
    pltpu.VMEM((_CH,), jnp.int32),
    pltpu.VMEM((_CH,), jnp.int32),
    pltpu.VMEM((_CH,), jnp.int32),
    pltpu.VMEM((_CH,), jnp.int32),
    pltpu.VMEM((_CH,), jnp.int32),
    pltpu.VMEM((_CH, _H), _f32),
    pltpu.VMEM((_CH, _H), _f32),
    pltpu.VMEM((16,), jnp.int32),
    pltpu.VMEM((16,), jnp.int32),
    pltpu.VMEM((16, _H), _f32),
    pltpu.VMEM((8,), jnp.int32),
    pltpu.VMEM((8,), jnp.int32),
    pltpu.VMEM((8, _H), _f32),
    pltpu.SemaphoreType.DMA,
    pltpu.SemaphoreType.DMA,
    pltpu.SemaphoreType.DMA,
    pltpu.SemaphoreType.DMA,
    pltpu.SemaphoreType.DMA,
    pltpu.SemaphoreType.DMA,
    pltpu.SemaphoreType.DMA,
    pltpu.SemaphoreType.DMA,
]


@functools.lru_cache(maxsize=None)
def _sc_kernels():
  mesh = plsc.VectorSubcoreMesh(
      core_axis_name="c", subcore_axis_name="s",
      num_cores=_NC, num_subcores=_NS,
  )
  scatter5 = pl.kernel(
      _sc_scatter5_body,
      out_type=jax.ShapeDtypeStruct((10 * _NP, _H), _f32),
      mesh=mesh,
      scratch_types=_SCR,
  )
  scatter4 = pl.kernel(
      _sc_scatter4_body,
      out_type=jax.ShapeDtypeStruct((8 * _NP, _H), _f32),
      mesh=mesh,
      scratch_types=_SCR,
  )
  return scatter5, scatter4


def _emb_body(x_ref, w_ref, b_ref, o_ref):
  o_ref[...] = (
      jnp.dot(x_ref[...], w_ref[...], preferred_element_type=_f32) + b_ref[...]
  )


_emb = pl.pallas_call(
    _emb_body,
    grid=(_GRID,),
    in_specs=[
        pl.BlockSpec((_BM, _DF), lambda m: (m, 0)),
        pl.BlockSpec((_DF, _H), lambda m: (0, 0)),
        pl.BlockSpec((1, _H), lambda m: (0, 0)),
    ],
    out_specs=pl.BlockSpec((_BM, _H), lambda m: (m, 0)),
    out_shape=jax.ShapeDtypeStruct((_N, _H), _f32),
)


def _sg_body(sgb_ref, h_ref, o_ref):
  m = pl.program_id(0)
  seg = sgb_ref[0, 0, :]
  oht = (lax.broadcasted_iota(jnp.int32, (_NSG, _BM), 0) == seg[None, :]
         ).astype(_f32)
  part = jnp.dot(oht, h_ref[...], preferred_element_type=_f32)

  @pl.when(m == 0)
  def _():
    o_ref[...] = part

  @pl.when(m > 0)
  def _():
    o_ref[...] += part


_sg_pool = pl.pallas_call(
    _sg_body,
    grid=(_GRID,),
    in_specs=[
        pl.BlockSpec((1, 1, _BM), lambda m: (m, 0, 0)),
        pl.BlockSpec((_BM, _H), lambda m: (m, 0)),
    ],
    out_specs=pl.BlockSpec((_NSG, _H), lambda m: (0, 0)),
    out_shape=jax.ShapeDtypeStruct((_NSG, _H), _f32),
)


def _layer_body(sgb_ref, h_ref, a_ref, ea_ref, sg_ref, wself_ref, wstk_ref,
                wea_ref, wpool_ref, b_ref, o_ref):
  acc = jnp.dot(h_ref[...], wself_ref[...], preferred_element_type=_f32)
  a = a_ref[...]
  for k in range(8):
    acc += jnp.dot(a[k], wstk_ref[k], preferred_element_type=_f32)
  ea = ea_ref[...]
  acc += jnp.dot(ea[0], wea_ref[0], preferred_element_type=_f32)
  acc += jnp.dot(ea[1], wea_ref[1], preferred_element_type=_f32)
  seg = sgb_ref[0, 0, :]
  oh = (seg[:, None] == lax.broadcasted_iota(jnp.int32, (_BM, _NSG), 1)
        ).astype(_f32)
  brd = jnp.dot(oh, sg_ref[...], preferred_element_type=_f32)
  acc += jnp.dot(brd, wpool_ref[...], preferred_element_type=_f32)
  o_ref[...] = jnp.maximum(acc + b_ref[...], 0.0)


_layer = pl.pallas_call(
    _layer_body,
    grid=(_GRID,),
    in_specs=[
        pl.BlockSpec((1, 1, _BM), lambda m: (m, 0, 0)),
        pl.BlockSpec((_BM, _H), lambda m: (m, 0)),
        pl.BlockSpec((8, _BM, _H), lambda m: (0, m, 0)),
        pl.BlockSpec((2, _BM, _H), lambda m: (0, m, 0)),
        pl.BlockSpec((_NSG, _H), lambda m: (0, 0)),
        pl.BlockSpec((_H, _H), lambda m: (0, 0)),
        pl.BlockSpec((8, _H, _H), lambda m: (0, 0, 0)),
        pl.BlockSpec((2, _H, _H), lambda m: (0, 0, 0)),
        pl.BlockSpec((_H, _H), lambda m: (0, 0)),
        pl.BlockSpec((1, _H), lambda m: (0, 0)),
    ],
    out_specs=pl.BlockSpec((_BM, _H), lambda m: (m, 0)),
    out_shape=jax.ShapeDtypeStruct((_N, _H), _f32),
)


def _final_body(bat_ref, h_ref, g_ref, w_ref, bo_ref, o_ref, accs):
  m = pl.program_id(0)
  seg = bat_ref[0, 0, :]
  g = g_ref[0, 0, :]
  hg = h_ref[...] * g[:, None]
  oht = (lax.broadcasted_iota(jnp.int32, (_NB, _BM), 0) == seg[None, :]
         ).astype(_f32)
  part = jnp.dot(oht, hg, preferred_element_type=_f32)

  @pl.when(m == 0)
  def _():
    accs[...] = part

  @pl.when(m > 0)
  def _():
    accs[...] += part

  @pl.when(m == _GRID - 1)
  def _():
    o_ref[...] = (
        jnp.dot(accs[...], w_ref[...], preferred_element_type=_f32)
        + bo_ref[...]
    )


_final = pl.pallas_call(
    _final_body,
    grid=(_GRID,),
    in_specs=[
        pl.BlockSpec((1, 1, _BM), lambda m: (m, 0, 0)),
        pl.BlockSpec((_BM, _H), lambda m: (m, 0)),
        pl.BlockSpec((1, 1, _BM), lambda m: (m, 0, 0)),
        pl.BlockSpec((_H, _OUTD), lambda m: (0, 0)),
        pl.BlockSpec((1, _OUTD), lambda m: (0, 0)),
    ],
    out_specs=pl.BlockSpec((_NB, _OUTD), lambda m: (0, 0)),
    out_shape=jax.ShapeDtypeStruct((_NB, _OUTD), _f32),
    scratch_shapes=[pltpu.VMEM((_NB, _OUTD), _f32)],
)


def kernel(x, edge_index, subgraph_edge_index, node_subnode_index,
           subnode_node_index, ground_node, subgraph_batch_index, batch_idx,
           edge_attr, W_emb, b_emb, W_self, W_e, W_sub, W_ns, W_sn, W_pool,
           b_mp, W_out, b_out):
  zeros_h = jnp.zeros((_ZB, _H), _f32)
  sgb3 = subgraph_batch_index.reshape(_GRID, 1, _BM)
  bat3 = batch_idx.reshape(_GRID, 1, _BM)
  g3 = ground_node.astype(_f32).reshape(_GRID, 1, _BM)
  s1, d1 = edge_index[0], edge_index[1]
  s2, d2 = subgraph_edge_index[0], subgraph_edge_index[1]
  s3, d3 = node_subnode_index[0], node_subnode_index[1]
  s4, d4 = subnode_node_index[0], subnode_node_index[1]

  scatter5, scatter4 = _sc_kernels()
  eidx = jnp.arange(_E, dtype=jnp.int32)
  ea_pad = jnp.pad(edge_attr, ((0, 0), (0, _H - _EDGE_F)))
  h = _emb(x, W_emb, b_emb.reshape(1, _H))

  ea = None
  for i in range(_DEPTH):
    if i == 0:
      out0 = scatter5(h, s1, d1, s2, d2, s3, d3, s4, d4, ea_pad, eidx,
                      zeros_h).reshape(10, _NP, _H)
      a = out0[:8]
      ea = out0[8:]
    else:
      a = scatter4(h, s1, d1, s2, d2, s3, d3, s4, d4,
                   zeros_h).reshape(8, _NP, _H)
    sg = _sg_pool(sgb3, h)
    weh = W_e[i, :_H, :]
    wstk = jnp.stack([weh, W_sub[i], W_ns[i], W_sn[i]] * 2)
    wea = jnp.stack([jnp.pad(W_e[i, _H:, :], ((0, _H - _EDGE_F), (0, 0)))] * 2)
    h = _layer(sgb3, h, a, ea, sg, W_self[i], wstk, wea, W_pool[i],
               b_mp[i].reshape(1, _H))

  return _final(bat3, h, g3, W_out, b_out.reshape(1, _OUTD))


# speedup vs baseline: 1.0550x; 1.0550x over previous
import functools

import jax
import jax.numpy as jnp
from jax import lax
from jax.experimental import pallas as pl
from jax.experimental.pallas import tpu as pltpu
from jax.experimental.pallas import tpu_sc as plsc

_N = 10000
_DF = 128
_H = 128
_OUTD = 128
_E = 320000
_ES = 320000
_ENS = 160000
_ESN = 160000
_EDGE_F = 16
_DEPTH = 2
_NSG = 512
_NB = 64

_BM = 400
_GRID = _N // _BM
_NC = 2
_NS = 16
_NW = _NC * _NS
_CH = 128
_RPT = 632
_NP = _RPT * _NS

_f32 = jnp.float32


_ZB = 64


def _zero_stripe(zbuf, dst, r0):
  nfull = _RPT // _ZB
  rz = _RPT % _ZB
  for q in range(nfull):
    pltpu.sync_copy(zbuf, dst.at[pl.ds(pl.multiple_of(r0 + q * _ZB, 8), _ZB)])
  if rz:
    pltpu.sync_copy(zbuf.at[pl.ds(0, rz)],
                    dst.at[pl.ds(pl.multiple_of(r0 + nfull * _ZB, 8), rz)])


def _flush_stripe(acc, out_hbm, r0, obase):
  nfull = _RPT // 128
  rz = _RPT % 128
  for q in range(nfull):
    pltpu.sync_copy(acc.at[pl.ds(pl.multiple_of(r0 + q * 128, 8), 128)],
                    out_hbm.at[pl.ds(pl.multiple_of(obase + r0 + q * 128, 8),
                                     128)])
  if rz:
    pltpu.sync_copy(
        acc.at[pl.ds(pl.multiple_of(r0 + nfull * 128, 8), rz)],
        out_hbm.at[pl.ds(pl.multiple_of(obase + r0 + nfull * 128, 8), rz)])


def _scatter_sets(out_hbm, sets, acc, zbuf, sidx, didx, rows,
                  sidx16, didx16, rows16, sidx8, didx8, rows8,
                  isem, gsem, ssem, c, s):
  r0 = s * _RPT

  for (table_hbm, se, de, cnt, obase) in sets:
    per = cnt // _NW
    base = c * (cnt // _NC) + s * per
    nfull = per // _CH
    rem = per % _CH
    nq = nfull // 4
    leftovers = list(range(4 * nq, nfull))

    def idx_load(j, ib):
      off = pl.multiple_of(base + j * _CH, 8)
      pltpu.async_copy(se.at[pl.ds(off, _CH)], sidx[ib], isem[ib])
      pltpu.async_copy(de.at[pl.ds(off, _CH)], didx[ib], isem[ib])

    def idx_wait(j, ib):
      off = pl.multiple_of(base + j * _CH, 8)
      pltpu.make_async_copy(se.at[pl.ds(off, _CH)], sidx[ib], isem[ib]).wait()
      pltpu.make_async_copy(de.at[pl.ds(off, _CH)], didx[ib], isem[ib]).wait()

    def gather(ib, rb):
      pltpu.async_copy(table_hbm.at[sidx[ib]], rows[rb], gsem[rb])

    def gather_wait(ib, rb):
      pltpu.make_async_copy(table_hbm.at[sidx[ib]], rows[rb], gsem[rb]).wait()

    def scat(ib, rb):
      pltpu.async_copy(rows[rb], acc.at[didx[ib]], ssem[rb], add=True)

    def scat_wait(ib, rb):
      pltpu.make_async_copy(rows[rb], acc.at[didx[ib]], ssem[rb]).wait()

    _zero_stripe(zbuf, acc, r0)
    plsc.subcore_barrier()

    idx_load(0, 0)
    idx_load(1, 1)

    def quad(q, carry):
      jq = 4 * q
      for b in range(4):
        j = jq + b
        ib = b
        rb = b % 2
        pib = (b + 3) % 4
        prb = (b + 1) % 2
        if b >= 2:
          scat_wait(b - 2, rb)
        else:
          @pl.when(q >= 1)
          def _():
            scat_wait((b + 2) % 4, rb)

        @pl.when(j + 2 < nfull)
        def _():
          idx_load(j + 2, (b + 2) % 4)

        idx_wait(j, ib)
        gather(ib, rb)
        if b == 0:
          @pl.when(q >= 1)
          def _():
            gather_wait(pib, prb)
            scat(pib, prb)
        else:
          gather_wait(pib, prb)
          scat(pib, prb)
      return carry

    lax.fori_loop(0, nq, quad, 0)

    for j in leftovers:
      ib = j % 4
      rb = j % 2
      if j >= 2:
        scat_wait((j - 2) % 4, rb)
      if j + 2 < nfull:
        idx_load(j + 2, (j + 2) % 4)
      idx_wait(j, ib)
      gather(ib, rb)
      if j >= 1:
        gather_wait((j - 1) % 4, (j - 1) % 2)
        scat((j - 1) % 4, (j - 1) % 2)
    gather_wait((nfull - 1) % 4, (nfull - 1) % 2)
    scat((nfull - 1) % 4, (nfull - 1) % 2)

    if rem:
      off = pl.multiple_of(base + nfull * _CH, 8)
      si, di, rw = (sidx16, didx16, rows16) if rem == 16 else (sidx8, didx8, rows8)
      pltpu.sync_copy(se.at[pl.ds(off, rem)], si)
      pltpu.sync_copy(de.at[pl.ds(off, rem)], di)
      pltpu.async_copy(table_hbm.at[si], rw, gsem[0]).wait()
      pltpu.sync_copy(rw, acc.at[di], add=True)

    scat_wait((nfull - 2) % 4, (nfull - 2) % 2)
    scat_wait((nfull - 1) % 4, (nfull - 1) % 2)

    plsc.subcore_barrier()
    _flush_stripe(acc, out_hbm, r0, obase)
    plsc.subcore_barrier()


def _sc_scatter_body(h_hbm, s1, d1, s2, d2, s3, d3, s4, d4, z_hbm, out_hbm,
                     acc, zbuf, si0, si1, si2, si3, di0, di1, di2, di3,
                     rows0, rows1, sidx16, didx16, rows16, sidx8, didx8, rows8,
                     is0, is1, is2, is3, gsem0, gsem1, ssem0, ssem1):
  c = lax.axis_index("c")
  s = lax.axis_index("s")
  pltpu.sync_copy(z_hbm, zbuf)
  sets = ((h_hbm, s1, d1, _E, (c * 4 + 0) * _NP),
          (h_hbm, s2, d2, _ES, (c * 4 + 1) * _NP),
          (h_hbm, s3, d3, _ENS, (c * 4 + 2) * _NP),
          (h_hbm, s4, d4, _ESN, (c * 4 + 3) * _NP))
  _scatter_sets(out_hbm, sets, acc, zbuf, (si0, si1, si2, si3),
                (di0, di1, di2, di3), (rows0, rows1),
                sidx16, didx16, rows16, sidx8, didx8, rows8,
                (is0, is1, is2, is3), (gsem0, gsem1), (ssem0, ssem1), c, s)


def _sc_ea_body(eap_hbm, eidx, d1, z_hbm, out_hbm,
                acc, zbuf, si0, si1, si2, si3, di0, di1, di2, di3,
                rows0, rows1, sidx16, didx16, rows16, sidx8, didx8, rows8,
                is0, is1, is2, is3, gsem0, gsem1, ssem0, ssem1):
  c = lax.axis_index("c")
  s = lax.axis_index("s")
  pltpu.sync_copy(z_hbm, zbuf)
  _scatter_sets(out_hbm, ((eap_hbm, eidx, d1, _E, c * _NP),), acc, zbuf,
                (si0, si1, si2, si3), (di0, di1, di2, di3), (rows0, rows1),
                sidx16, didx16, rows16, sidx8, didx8, rows8,
                (is0, is1, is2, is3), (gsem0, gsem1), (ssem0, ssem1), c, s)


_SCR = [
    pltpu.VMEM_SHARED((_NP, _H), _f32),
    pltpu.VMEM((_ZB, _H), _f32),
    pltpu.VMEM((_CH,), jnp.int32),
    pltpu.VMEM((_CH,), jnp.int32),
    pltpu.VMEM((_CH,), jnp.int32),
    pltpu.VMEM((_CH,), jnp.int32),
    pltpu.VMEM((_CH,), jnp.int32),
    pltpu.VMEM((_CH,), jnp.int32),
    pltpu.VMEM((_CH,), jnp.int32),
    pltpu.VMEM((_CH,), jnp.int32),
    pltpu.VMEM((_CH, _H), _f32),
    pltpu.VMEM((_CH, _H), _f32),
    pltpu.VMEM((16,), jnp.int32),
    pltpu.VMEM((16,), jnp.int32),
    pltpu.VMEM((16, _H), _f32),
    pltpu.VMEM((8,), jnp.int32),
    pltpu.VMEM((8,), jnp.int32),
    pltpu.VMEM((8, _H), _f32),
    pltpu.SemaphoreType.DMA,
    pltpu.SemaphoreType.DMA,
    pltpu.SemaphoreType.DMA,
    pltpu.SemaphoreType.DMA,
    pltpu.SemaphoreType.DMA,
    pltpu.SemaphoreType.DMA,
    pltpu.SemaphoreType.DMA,
    pltpu.SemaphoreType.DMA,
]


@functools.lru_cache(maxsize=None)
def _sc_kernels():
  mesh = plsc.VectorSubcoreMesh(
      core_axis_name="c", subcore_axis_name="s",
      num_cores=_NC, num_subcores=_NS,
  )
  scatter = pl.kernel(
      _sc_scatter_body,
      out_type=jax.ShapeDtypeStruct((8 * _NP, _H), _f32),
      mesh=mesh,
      scratch_types=_SCR,
  )
  ea = pl.kernel(
      _sc_ea_body,
      out_type=jax.ShapeDtypeStruct((2 * _NP, _H), _f32),
      mesh=mesh,
      scratch_types=_SCR,
  )
  return scatter, ea


def _emb_body(x_ref, w_ref, b_ref, o_ref):
  o_ref[...] = (
      jnp.dot(x_ref[...], w_ref[...], preferred_element_type=_f32) + b_ref[...]
  )


_emb = pl.pallas_call(
    _emb_body,
    grid=(_GRID,),
    in_specs=[
        pl.BlockSpec((_BM, _DF), lambda m: (m, 0)),
        pl.BlockSpec((_DF, _H), lambda m: (0, 0)),
        pl.BlockSpec((1, _H), lambda m: (0, 0)),
    ],
    out_specs=pl.BlockSpec((_BM, _H), lambda m: (m, 0)),
    out_shape=jax.ShapeDtypeStruct((_N, _H), _f32),
)


def _sg_body(sgb_ref, h_ref, o_ref):
  m = pl.program_id(0)
  seg = sgb_ref[0, 0, :]
  oht = (lax.broadcasted_iota(jnp.int32, (_NSG, _BM), 0) == seg[None, :]
         ).astype(_f32)
  part = jnp.dot(oht, h_ref[...], preferred_element_type=_f32)

  @pl.when(m == 0)
  def _():
    o_ref[...] = part

  @pl.when(m > 0)
  def _():
    o_ref[...] += part


_sg_pool = pl.pallas_call(
    _sg_body,
    grid=(_GRID,),
    in_specs=[
        pl.BlockSpec((1, 1, _BM), lambda m: (m, 0, 0)),
        pl.BlockSpec((_BM, _H), lambda m: (m, 0)),
    ],
    out_specs=pl.BlockSpec((_NSG, _H), lambda m: (0, 0)),
    out_shape=jax.ShapeDtypeStruct((_NSG, _H), _f32),
)


def _layer_body(sgb_ref, h_ref, a_ref, ea_ref, sg_ref, wself_ref, wstk_ref,
                wea_ref, wpool_ref, b_ref, o_ref):
  acc = jnp.dot(h_ref[...], wself_ref[...], preferred_element_type=_f32)
  a = a_ref[...]
  for k in range(8):
    acc += jnp.dot(a[k], wstk_ref[k], preferred_element_type=_f32)
  ea = ea_ref[...]
  acc += jnp.dot(ea[0], wea_ref[0], preferred_element_type=_f32)
  acc += jnp.dot(ea[1], wea_ref[1], preferred_element_type=_f32)
  seg = sgb_ref[0, 0, :]
  oh = (seg[:, None] == lax.broadcasted_iota(jnp.int32, (_BM, _NSG), 1)
        ).astype(_f32)
  brd = jnp.dot(oh, sg_ref[...], preferred_element_type=_f32)
  acc += jnp.dot(brd, wpool_ref[...], preferred_element_type=_f32)
  o_ref[...] = jnp.maximum(acc + b_ref[...], 0.0)


_layer = pl.pallas_call(
    _layer_body,
    grid=(_GRID,),
    in_specs=[
        pl.BlockSpec((1, 1, _BM), lambda m: (m, 0, 0)),
        pl.BlockSpec((_BM, _H), lambda m: (m, 0)),
        pl.BlockSpec((8, _BM, _H), lambda m: (0, m, 0)),
        pl.BlockSpec((2, _BM, _H), lambda m: (0, m, 0)),
        pl.BlockSpec((_NSG, _H), lambda m: (0, 0)),
        pl.BlockSpec((_H, _H), lambda m: (0, 0)),
        pl.BlockSpec((8, _H, _H), lambda m: (0, 0, 0)),
        pl.BlockSpec((2, _H, _H), lambda m: (0, 0, 0)),
        pl.BlockSpec((_H, _H), lambda m: (0, 0)),
        pl.BlockSpec((1, _H), lambda m: (0, 0)),
    ],
    out_specs=pl.BlockSpec((_BM, _H), lambda m: (m, 0)),
    out_shape=jax.ShapeDtypeStruct((_N, _H), _f32),
)


def _final_body(bat_ref, h_ref, g_ref, w_ref, bo_ref, o_ref, accs):
  m = pl.program_id(0)
  seg = bat_ref[0, 0, :]
  g = g_ref[0, 0, :]
  hg = h_ref[...] * g[:, None]
  oht = (lax.broadcasted_iota(jnp.int32, (_NB, _BM), 0) == seg[None, :]
         ).astype(_f32)
  part = jnp.dot(oht, hg, preferred_element_type=_f32)

  @pl.when(m == 0)
  def _():
    accs[...] = part

  @pl.when(m > 0)
  def _():
    accs[...] += part

  @pl.when(m == _GRID - 1)
  def _():
    o_ref[...] = (
        jnp.dot(accs[...], w_ref[...], preferred_element_type=_f32)
        + bo_ref[...]
    )


_final = pl.pallas_call(
    _final_body,
    grid=(_GRID,),
    in_specs=[
        pl.BlockSpec((1, 1, _BM), lambda m: (m, 0, 0)),
        pl.BlockSpec((_BM, _H), lambda m: (m, 0)),
        pl.BlockSpec((1, 1, _BM), lambda m: (m, 0, 0)),
        pl.BlockSpec((_H, _OUTD), lambda m: (0, 0)),
        pl.BlockSpec((1, _OUTD), lambda m: (0, 0)),
    ],
    out_specs=pl.BlockSpec((_NB, _OUTD), lambda m: (0, 0)),
    out_shape=jax.ShapeDtypeStruct((_NB, _OUTD), _f32),
    scratch_shapes=[pltpu.VMEM((_NB, _OUTD), _f32)],
)


def kernel(x, edge_index, subgraph_edge_index, node_subnode_index,
           subnode_node_index, ground_node, subgraph_batch_index, batch_idx,
           edge_attr, W_emb, b_emb, W_self, W_e, W_sub, W_ns, W_sn, W_pool,
           b_mp, W_out, b_out):
  zeros_h = jnp.zeros((_ZB, _H), _f32)
  sgb3 = subgraph_batch_index.reshape(_GRID, 1, _BM)
  bat3 = batch_idx.reshape(_GRID, 1, _BM)
  g3 = ground_node.astype(_f32).reshape(_GRID, 1, _BM)
  s1, d1 = edge_index[0], edge_index[1]
  s2, d2 = subgraph_edge_index[0], subgraph_edge_index[1]
  s3, d3 = node_subnode_index[0], node_subnode_index[1]
  s4, d4 = subnode_node_index[0], subnode_node_index[1]

  sc_scatter, sc_ea = _sc_kernels()
  eidx = jnp.arange(_E, dtype=jnp.int32)
  ea_pad = jnp.pad(edge_attr, ((0, 0), (0, _H - _EDGE_F)))
  ea_flat = sc_ea(ea_pad, eidx, d1, zeros_h)
  h = _emb(x, W_emb, b_emb.reshape(1, _H))
  h, ea_flat = lax.optimization_barrier((h, ea_flat))
  ea = ea_flat.reshape(2, _NP, _H)

  for i in range(_DEPTH):
    a = sc_scatter(h, s1, d1, s2, d2, s3, d3, s4, d4,
                   zeros_h).reshape(8, _NP, _H)
    sg = _sg_pool(sgb3, h)
    weh = W_e[i, :_H, :]
    wstk = jnp.stack([weh, W_sub[i], W_ns[i], W_sn[i]] * 2)
    wea = jnp.stack([jnp.pad(W_e[i, _H:, :], ((0, _H - _EDGE_F), (0, 0)))] * 2)
    h = _layer(sgb3, h, a, ea, sg, W_self[i], wstk, wea, W_pool[i],
               b_mp[i].reshape(1, _H))

  return _final(bat3, h, g3, W_out, b_out.reshape(1, _OUTD))
